# baseline (speedup 1.0000x reference)
import functools

import jax
import jax.numpy as jnp
from jax import lax
from jax.experimental import pallas as pl
from jax.experimental.pallas import tpu as pltpu
from jax.experimental.pallas import tpu_sc as plsc

H = 128
CH = 128
NC, NS = 2, 16
NW = NC * NS

_MESH = dict(core_axis_name="c", subcore_axis_name="s")


def _wid():
    return lax.axis_index("s") * NC + lax.axis_index("c")


def _bdot(x, w):
    return jnp.dot(x.astype(jnp.bfloat16), w.astype(jnp.bfloat16),
                   preferred_element_type=jnp.float32)


def _chunk_off(g, nrows):
    return jnp.minimum(g * CH, nrows - CH)


def _egather_batched(tab_hbm, iv, rv, sem, nidx, clamp=None):
    nstream = nidx // CH
    FB = 8
    for t0 in range(0, nstream, FB):
        hs = []
        for b in range(min(FB, nstream - t0)):
            off = (t0 + b) * CH
            src_idx = iv.at[pl.ds(off, CH)]
            hs.append(pltpu.async_copy(tab_hbm.at[src_idx],
                                       rv.at[pl.ds(off, CH)], sem))
        for h in hs:
            h.wait()


def _gather_mn(emb, mnemic, mnidx):
    B = mnidx.shape[0]
    RPW = B // NW
    NB = RPW // 256
    mesh = plsc.VectorSubcoreMesh(**_MESH)

    @functools.partial(
        pl.kernel, mesh=mesh,
        out_type=jax.ShapeDtypeStruct((B, H), jnp.float32),
        scratch_types=[
            pltpu.VMEM((RPW,), jnp.int32),
            pltpu.VMEM((RPW,), jnp.int32),
            pltpu.VMEM((256, H), jnp.float32),
            pltpu.VMEM((256, H), jnp.float32),
            pltpu.SemaphoreType.DMA,
            pltpu.SemaphoreType.DMA,
        ],
    )
    def k(emb_hbm, mn_hbm, idx_hbm, out_hbm, iv, mv, rows0, rows1, s1, s2):
        w = _wid()
        base = w * RPW
        pltpu.sync_copy(idx_hbm.at[pl.ds(base, RPW)], iv)
        _egather_batched(mn_hbm, iv, mv, s1, RPW)

        def body(t, carry):
            hs = []
            for half, rows in ((0, rows0), (1, rows1)):
                o = (2 * t + half) * 256
                hs.append(pltpu.async_copy(
                    emb_hbm.at[mv.at[pl.ds(o, CH)]],
                    rows.at[pl.ds(0, CH)], s2))
                hs.append(pltpu.async_copy(
                    emb_hbm.at[mv.at[pl.ds(o + CH, CH)]],
                    rows.at[pl.ds(CH, CH)], s2))
            for half, rows in ((0, rows0), (1, rows1)):
                o = (2 * t + half) * 256
                hs[2 * half].wait()
                hs[2 * half + 1].wait()
                pltpu.sync_copy(rows, out_hbm.at[pl.ds(base + o, 256)])
            return carry
        lax.fori_loop(0, NB // 2, body, 0)

    return k(emb, mnemic, mnidx)


def _gather_mem(emb, op_imm, regs, mr0, mr1, mi0, mi1):
    B = mr0.shape[0]
    SPAN = ((-(-B // NW)) + 7) // 8 * 8
    NST = (SPAN + CH - 1) // CH
    mesh = plsc.VectorSubcoreMesh(**_MESH)
    sds = jax.ShapeDtypeStruct((B, H), jnp.float32)

    @functools.partial(
        pl.kernel, mesh=mesh,
        out_type=(sds, sds, sds, sds),
        scratch_types=[
            pltpu.VMEM((SPAN,), jnp.int32),
            pltpu.VMEM((SPAN,), jnp.int32),
            pltpu.VMEM((SPAN,), jnp.int32),
            pltpu.VMEM((SPAN,), jnp.int32),
            pltpu.VMEM((SPAN,), jnp.int32),
            pltpu.VMEM((SPAN,), jnp.int32),
            pltpu.VMEM((CH, H), jnp.float32),
            pltpu.VMEM((CH, H), jnp.float32),
            pltpu.VMEM((CH, H), jnp.float32),
            pltpu.VMEM((CH, H), jnp.float32),
            pltpu.SemaphoreType.DMA,
            pltpu.SemaphoreType.DMA,
            pltpu.SemaphoreType.DMA,
        ],
    )
    def k(emb_hbm, imm_hbm, regs_hbm, mr0_hbm, mr1_hbm, mi0_hbm, mi1_hbm,
          a_hbm, b_hbm, c_hbm, d_hbm,
          iv0, iv1, iv2, iv3, rv0, rv1, ra, rb, rc, rd, s1, s2, s3):
        w = _wid()
        base = jnp.minimum(w * SPAN, B - SPAN)
        hs = [pltpu.async_copy(src.at[pl.ds(base, SPAN)], dst, s1)
              for src, dst in ((mr0_hbm, iv0), (mr1_hbm, iv1),
                               (mi0_hbm, iv2), (mi1_hbm, iv3))]
        for h in hs:
            h.wait()
        pairs = [(iv, rv, c) for c in range(NST) for iv, rv in
                 ((iv0, rv0), (iv1, rv1))]
        for t0 in range(0, len(pairs), 8):
            hs = []
            for iv, rv, c in pairs[t0:t0 + 8]:
                o = min(c * CH, SPAN - CH)
                hs.append(pltpu.async_copy(
                    regs_hbm.at[iv.at[pl.ds(o, CH)]],
                    rv.at[pl.ds(o, CH)], s1))
            for h in hs:
                h.wait()

        def body(c, carry):
            o = jnp.minimum(c * CH, SPAN - CH)
            hs = [pltpu.async_copy(tab.at[idx.at[pl.ds(o, CH)]], rows, s2)
                  for tab, idx, rows in ((emb_hbm, rv0, ra), (emb_hbm, rv1, rb),
                                         (imm_hbm, iv2, rc), (imm_hbm, iv3, rd))]
            for h in hs:
                h.wait()
            ws = [pltpu.async_copy(rows, dst.at[pl.ds(base + o, CH)], s3)
                  for rows, dst in ((ra, a_hbm), (rb, b_hbm),
                                    (rc, c_hbm), (rd, d_hbm))]
            for h in ws:
                h.wait()
            return carry
        lax.fori_loop(0, NST, body, 0)

    return k(emb, op_imm, regs, mr0, mr1, mi0, mi1)


def _gather_final(table, regs_ext, opidx):
    B = opidx.shape[0]
    mesh = plsc.VectorSubcoreMesh(**_MESH)
    RPW = B // NW
    NB = RPW // 256

    @functools.partial(
        pl.kernel, mesh=mesh,
        out_type=jax.ShapeDtypeStruct((B, H), jnp.float32),
        scratch_types=[
            pltpu.VMEM((RPW,), jnp.int32),
            pltpu.VMEM((RPW,), jnp.int32),
            pltpu.VMEM((256, H), jnp.float32),
            pltpu.VMEM((256, H), jnp.float32),
            pltpu.SemaphoreType.DMA,
            pltpu.SemaphoreType.DMA,
        ],
    )
    def k(tab_hbm, rext_hbm, idx_hbm, out_hbm, iv, cv, rows0, rows1, s1, s2):
        w = _wid()
        base = w * RPW
        pltpu.sync_copy(idx_hbm.at[pl.ds(base, RPW)], iv)
        _egather_batched(rext_hbm, iv, cv, s1, RPW)

        def body(t, carry):
            hs = []
            for half, rows in ((0, rows0), (1, rows1)):
                o = (2 * t + half) * 256
                hs.append(pltpu.async_copy(
                    tab_hbm.at[cv.at[pl.ds(o, CH)]],
                    rows.at[pl.ds(0, CH)], s2))
                hs.append(pltpu.async_copy(
                    tab_hbm.at[cv.at[pl.ds(o + CH, CH)]],
                    rows.at[pl.ds(CH, CH)], s2))
            for half, rows in ((0, rows0), (1, rows1)):
                o = (2 * t + half) * 256
                hs[2 * half].wait()
                hs[2 * half + 1].wait()
                pltpu.sync_copy(rows, out_hbm.at[pl.ds(base + o, 256)])
            return carry
        lax.fori_loop(0, NB // 2, body, 0)

    return k(table, regs_ext, opidx)


def _imm_mlp(imm, w1, b1, w2, b2):
    N = imm.shape[0]
    BLK = 5000
    nblk = N // BLK

    def body(x_ref, w1_ref, b1_ref, w2_ref, b2_ref, o_ref):
        h = jnp.maximum(jnp.tanh(x_ref[...]) * w1_ref[...] + b1_ref[...], 0.0)
        o_ref[...] = _bdot(h, w2_ref[...]) + b2_ref[...]

    return pl.pallas_call(
        body,
        grid=(nblk,),
        in_specs=[
            pl.BlockSpec((BLK, 1), lambda i: (i, 0)),
            pl.BlockSpec((1, H), lambda i: (0, 0)),
            pl.BlockSpec((1, H), lambda i: (0, 0)),
            pl.BlockSpec((H, H), lambda i: (0, 0)),
            pl.BlockSpec((1, H), lambda i: (0, 0)),
        ],
        out_specs=pl.BlockSpec((BLK, H), lambda i: (i, 0)),
        out_shape=jax.ShapeDtypeStruct((N, H), jnp.float32),
    )(imm, w1, b1.reshape(1, H), w2, b2.reshape(1, H))


def _w_spec(shp):
    return pl.BlockSpec(shp, lambda i: (0, 0))


def _imm_into_table(op_imm, ntab, row0):
    BLK = 2000
    nblk = op_imm.shape[0] // BLK
    blk0 = row0 // BLK

    def body(x_ref, o_ref):
        o_ref[...] = x_ref[...]

    return pl.pallas_call(
        body,
        grid=(nblk,),
        in_specs=[pl.BlockSpec((BLK, H), lambda i: (i, 0))],
        out_specs=pl.BlockSpec((BLK, H), lambda i: (i + blk0, 0)),
        out_shape=jax.ShapeDtypeStruct((ntab, H), jnp.float32),
    )(op_imm)


def _reg_table(table, emb, wr1, br1, wr2, br2):
    BLK = 2000
    nblk = emb.shape[0] // BLK

    def body(tab_ref, emb_ref, wr1_ref, br1_ref, wr2_ref, br2_ref, t_ref):
        h = jnp.maximum(_bdot(emb_ref[...], wr1_ref[...]) + br1_ref[...], 0.0)
        t_ref[...] = _bdot(h, wr2_ref[...]) + br2_ref[...]

    return pl.pallas_call(
        body,
        grid=(nblk,),
        in_specs=[
            pl.BlockSpec(memory_space=pl.ANY),
            pl.BlockSpec((BLK, H), lambda i: (i, 0)),
            _w_spec((H, H)), _w_spec((1, H)), _w_spec((H, H)), _w_spec((1, H)),
        ],
        out_specs=pl.BlockSpec((BLK, H), lambda i: (i, 0)),
        out_shape=jax.ShapeDtypeStruct(table.shape, jnp.float32),
        input_output_aliases={0: 0},
    )(table, emb, wr1, br1.reshape(1, H), wr2, br2.reshape(1, H))


def _mem_into_table(table, a, b, c, d, wm1, bm1, wm2, bm2, row0):
    BLK = 2000
    nblk = a.shape[0] // BLK
    blk0 = row0 // BLK

    def body(tab_ref, a_ref, b_ref, c_ref, d_ref,
             wm1_ref, bm1_ref, wm2_ref, bm2_ref, o_ref):
        acc = (_bdot(a_ref[...], wm1_ref[0:H])
               + _bdot(b_ref[...], wm1_ref[H:2 * H])
               + _bdot(c_ref[...], wm1_ref[2 * H:3 * H])
               + _bdot(d_ref[...], wm1_ref[3 * H:4 * H])
               + bm1_ref[...])
        h = jnp.maximum(acc, 0.0)
        o_ref[...] = _bdot(h, wm2_ref[...]) + bm2_ref[...]

    blkspec = lambda: pl.BlockSpec((BLK, H), lambda i: (i, 0))
    return pl.pallas_call(
        body,
        grid=(nblk,),
        in_specs=[
            pl.BlockSpec(memory_space=pl.ANY),
            blkspec(), blkspec(), blkspec(), blkspec(),
            _w_spec((4 * H, H)), _w_spec((1, H)), _w_spec((H, H)), _w_spec((1, H)),
        ],
        out_specs=pl.BlockSpec((BLK, H), lambda i: (i + blk0, 0)),
        out_shape=jax.ShapeDtypeStruct(table.shape, jnp.float32),
        input_output_aliases={0: 0},
    )(table, a, b, c, d,
      wm1, bm1.reshape(1, H), wm2, bm2.reshape(1, H))


def _final_mlp_half(prev_out, mn_g, g_half, w1, b1, w2, b2,
                    bsz, slen, half, nhalves):
    RB = 64
    BLK = RB * slen
    nblk = bsz // nhalves // RB
    mn0 = half * nblk

    def body(tab_ref, mn_ref, g0_ref, g1_ref, g2_ref, g3_ref,
             w1_ref, b1_ref, w2_ref, b2_ref, o_ref):
        acc = (_bdot(mn_ref[...], w1_ref[0:H])
               + _bdot(g0_ref[...], w1_ref[H:2 * H])
               + _bdot(g1_ref[...], w1_ref[2 * H:3 * H])
               + _bdot(g2_ref[...], w1_ref[3 * H:4 * H])
               + _bdot(g3_ref[...], w1_ref[4 * H:5 * H])
               + b1_ref[...])
        h = jnp.maximum(acc, 0.0)
        o_ref[...] = (_bdot(h, w2_ref[...]) + b2_ref[...]).reshape(RB, slen, H)

    slot = lambda k: pl.BlockSpec((BLK, H), lambda i, k=k: (k * nblk + i, 0))
    first = prev_out is None
    args = [mn_g, g_half, g_half, g_half, g_half,
            w1, b1.reshape(1, H), w2, b2.reshape(1, H)]
    in_specs = [
        pl.BlockSpec((BLK, H), lambda i: (mn0 + i, 0)),
        slot(0), slot(1), slot(2), slot(3),
        _w_spec((5 * H, H)), _w_spec((1, H)), _w_spec((H, H)), _w_spec((1, H)),
    ]
    if first:
        def body1(*refs):
            body(None, *refs)
        kernel_body = body1
        aliases = {}
    else:
        args = [prev_out] + args
        in_specs = [pl.BlockSpec(memory_space=pl.ANY)] + in_specs
        kernel_body = body
        aliases = {0: 0}
    return pl.pallas_call(
        kernel_body,
        grid=(nblk,),
        in_specs=in_specs,
        out_specs=pl.BlockSpec((RB, slen, H), lambda i: (mn0 + i, 0, 0)),
        out_shape=jax.ShapeDtypeStruct((bsz, slen, H), jnp.float32),
        input_output_aliases=aliases,
    )(*args)


def kernel(imm, regs, mem_reg0, mem_reg1, mem_imm0, mem_imm1, mnemic,
           mnemic_idx, operand_idx, emb, W_imm1, b_imm1, W_imm2, b_imm2,
           W_reg1, b_reg1, W_reg2, b_reg2, W_mem1, b_mem1, W_mem2, b_mem2,
           W_ins1, b_ins1, W_ins2, b_ins2):
    nreg = regs.shape[0]
    nimm = imm.shape[0]
    nmem = mem_reg0.shape[0]
    ntab = nreg + nimm + nmem
    regs = regs.astype(jnp.int32)
    bsz, slen = mnemic_idx.shape

    regs_ext = jnp.concatenate(
        [regs, jnp.arange(nreg, ntab, dtype=jnp.int32)])
    nh = bsz // 2 * slen
    oi = operand_idx.reshape(-1, 4)
    opcats = [oi[h * nh:(h + 1) * nh].T.reshape(-1).astype(jnp.int32)
              for h in (0, 1)]
    regs_ext, opcats[0], opcats[1], imm, emb = lax.optimization_barrier(
        (regs_ext, opcats[0], opcats[1], imm, emb))

    op_imm = _imm_mlp(imm, W_imm1, b_imm1, W_imm2, b_imm2)
    mn_g = _gather_mn(emb, mnemic.astype(jnp.int32),
                      mnemic_idx.reshape(-1).astype(jnp.int32))
    a, b, c, d = _gather_mem(emb, op_imm, regs,
                             mem_reg0.astype(jnp.int32),
                             mem_reg1.astype(jnp.int32),
                             mem_imm0.astype(jnp.int32),
                             mem_imm1.astype(jnp.int32))
    table1 = _imm_into_table(op_imm, ntab, nreg)
    table2 = _reg_table(table1, emb, W_reg1, b_reg1, W_reg2, b_reg2)
    table3 = _mem_into_table(table2, a, b, c, d,
                             W_mem1, b_mem1, W_mem2, b_mem2, nreg + nimm)
    out = None
    gs = []
    for half in (0, 1):
        gs.append(_gather_final(table3, regs_ext, opcats[half]))
    for half in (0, 1):
        out = _final_mlp_half(out, mn_g, gs[half], W_ins1, b_ins1,
                              W_ins2, b_ins2, bsz, slen, half, 2)
    return out


# speedup vs baseline: 1.0405x; 1.0405x over previous
import functools

import jax
import jax.numpy as jnp
from jax import lax
from jax.experimental import pallas as pl
from jax.experimental.pallas import tpu as pltpu
from jax.experimental.pallas import tpu_sc as plsc

H = 128
CH = 128
NC, NS = 2, 16
NW = NC * NS

_MESH = dict(core_axis_name="c", subcore_axis_name="s")


def _wid():
    return lax.axis_index("s") * NC + lax.axis_index("c")


def _bdot(x, w):
    return jnp.dot(x.astype(jnp.bfloat16), w.astype(jnp.bfloat16),
                   preferred_element_type=jnp.float32)


def _chunk_off(g, nrows):
    return jnp.minimum(g * CH, nrows - CH)


def _egather_batched(tab_hbm, iv, rv, sem, nidx, clamp=None):
    nstream = nidx // CH
    FB = 8
    for t0 in range(0, nstream, FB):
        hs = []
        for b in range(min(FB, nstream - t0)):
            off = (t0 + b) * CH
            src_idx = iv.at[pl.ds(off, CH)]
            hs.append(pltpu.async_copy(tab_hbm.at[src_idx],
                                       rv.at[pl.ds(off, CH)], sem))
        for h in hs:
            h.wait()


def _gather_mn(emb, mnemic, mnidx):
    B = mnidx.shape[0]
    RPW = B // NW
    NB = RPW // 256
    mesh = plsc.VectorSubcoreMesh(**_MESH)

    @functools.partial(
        pl.kernel, mesh=mesh,
        out_type=jax.ShapeDtypeStruct((B, H), jnp.float32),
        scratch_types=[
            pltpu.VMEM((RPW,), jnp.int32),
            pltpu.VMEM((RPW,), jnp.int32),
            pltpu.VMEM((256, H), jnp.float32),
            pltpu.VMEM((256, H), jnp.float32),
            pltpu.SemaphoreType.DMA,
            pltpu.SemaphoreType.DMA,
        ],
    )
    def k(emb_hbm, mn_hbm, idx_hbm, out_hbm, iv, mv, rows0, rows1, s1, s2):
        w = _wid()
        base = w * RPW
        pltpu.sync_copy(idx_hbm.at[pl.ds(base, RPW)], iv)
        _egather_batched(mn_hbm, iv, mv, s1, RPW)

        def body(t, carry):
            hs = []
            for half, rows in ((0, rows0), (1, rows1)):
                o = (2 * t + half) * 256
                hs.append(pltpu.async_copy(
                    emb_hbm.at[mv.at[pl.ds(o, CH)]],
                    rows.at[pl.ds(0, CH)], s2))
                hs.append(pltpu.async_copy(
                    emb_hbm.at[mv.at[pl.ds(o + CH, CH)]],
                    rows.at[pl.ds(CH, CH)], s2))
            for half, rows in ((0, rows0), (1, rows1)):
                o = (2 * t + half) * 256
                hs[2 * half].wait()
                hs[2 * half + 1].wait()
                pltpu.sync_copy(rows, out_hbm.at[pl.ds(base + o, 256)])
            return carry
        lax.fori_loop(0, NB // 2, body, 0)

    return k(emb, mnemic, mnidx)


def _gather_mem(emb, op_imm, regs, mr0, mr1, mi0, mi1):
    B = mr0.shape[0]
    SPAN = ((-(-B // NW)) + 7) // 8 * 8
    NST = (SPAN + CH - 1) // CH
    mesh = plsc.VectorSubcoreMesh(**_MESH)
    sds = jax.ShapeDtypeStruct((B, H), jnp.float32)

    @functools.partial(
        pl.kernel, mesh=mesh,
        out_type=(sds, sds, sds, sds),
        scratch_types=[
            pltpu.VMEM((SPAN,), jnp.int32),
            pltpu.VMEM((SPAN,), jnp.int32),
            pltpu.VMEM((SPAN,), jnp.int32),
            pltpu.VMEM((SPAN,), jnp.int32),
            pltpu.VMEM((SPAN,), jnp.int32),
            pltpu.VMEM((SPAN,), jnp.int32),
            pltpu.VMEM((CH, H), jnp.float32),
            pltpu.VMEM((CH, H), jnp.float32),
            pltpu.VMEM((CH, H), jnp.float32),
            pltpu.VMEM((CH, H), jnp.float32),
            pltpu.SemaphoreType.DMA,
            pltpu.SemaphoreType.DMA,
            pltpu.SemaphoreType.DMA,
        ],
    )
    def k(emb_hbm, imm_hbm, regs_hbm, mr0_hbm, mr1_hbm, mi0_hbm, mi1_hbm,
          a_hbm, b_hbm, c_hbm, d_hbm,
          iv0, iv1, iv2, iv3, rv0, rv1, ra, rb, rc, rd, s1, s2, s3):
        w = _wid()
        base = jnp.minimum(w * SPAN, B - SPAN)
        hs = [pltpu.async_copy(src.at[pl.ds(base, SPAN)], dst, s1)
              for src, dst in ((mr0_hbm, iv0), (mr1_hbm, iv1),
                               (mi0_hbm, iv2), (mi1_hbm, iv3))]
        for h in hs:
            h.wait()
        pairs = [(iv, rv, c) for c in range(NST) for iv, rv in
                 ((iv0, rv0), (iv1, rv1))]
        for t0 in range(0, len(pairs), 8):
            hs = []
            for iv, rv, c in pairs[t0:t0 + 8]:
                o = min(c * CH, SPAN - CH)
                hs.append(pltpu.async_copy(
                    regs_hbm.at[iv.at[pl.ds(o, CH)]],
                    rv.at[pl.ds(o, CH)], s1))
            for h in hs:
                h.wait()

        def body(c, carry):
            o = jnp.minimum(c * CH, SPAN - CH)
            hs = [pltpu.async_copy(tab.at[idx.at[pl.ds(o, CH)]], rows, s2)
                  for tab, idx, rows in ((emb_hbm, rv0, ra), (emb_hbm, rv1, rb),
                                         (imm_hbm, iv2, rc), (imm_hbm, iv3, rd))]
            for h in hs:
                h.wait()
            ws = [pltpu.async_copy(rows, dst.at[pl.ds(base + o, CH)], s3)
                  for rows, dst in ((ra, a_hbm), (rb, b_hbm),
                                    (rc, c_hbm), (rd, d_hbm))]
            for h in ws:
                h.wait()
            return carry
        lax.fori_loop(0, NST, body, 0)

    return k(emb, op_imm, regs, mr0, mr1, mi0, mi1)


def _gather_final(table, regs_ext, opidx):
    B = opidx.shape[0]
    mesh = plsc.VectorSubcoreMesh(**_MESH)
    RPW = B // NW
    NB = RPW // 256

    @functools.partial(
        pl.kernel, mesh=mesh,
        out_type=jax.ShapeDtypeStruct((B, H), jnp.float32),
        scratch_types=[
            pltpu.VMEM((RPW,), jnp.int32),
            pltpu.VMEM((RPW,), jnp.int32),
            pltpu.VMEM((256, H), jnp.float32),
            pltpu.VMEM((256, H), jnp.float32),
            pltpu.SemaphoreType.DMA,
            pltpu.SemaphoreType.DMA,
        ],
    )
    def k(tab_hbm, rext_hbm, idx_hbm, out_hbm, iv, cv, rows0, rows1, s1, s2):
        w = _wid()
        base = w * RPW
        pltpu.sync_copy(idx_hbm.at[pl.ds(base, RPW)], iv)
        _egather_batched(rext_hbm, iv, cv, s1, RPW)

        def body(t, carry):
            hs = []
            for half, rows in ((0, rows0), (1, rows1)):
                o = (2 * t + half) * 256
                hs.append(pltpu.async_copy(
                    tab_hbm.at[cv.at[pl.ds(o, CH)]],
                    rows.at[pl.ds(0, CH)], s2))
                hs.append(pltpu.async_copy(
                    tab_hbm.at[cv.at[pl.ds(o + CH, CH)]],
                    rows.at[pl.ds(CH, CH)], s2))
            for half, rows in ((0, rows0), (1, rows1)):
                o = (2 * t + half) * 256
                hs[2 * half].wait()
                hs[2 * half + 1].wait()
                pltpu.sync_copy(rows, out_hbm.at[pl.ds(base + o, 256)])
            return carry
        lax.fori_loop(0, NB // 2, body, 0)

    return k(table, regs_ext, opidx)


def _imm_mlp(imm, w1, b1, w2, b2):
    N = imm.shape[0]
    BLK = 5000
    nblk = N // BLK

    def body(x_ref, w1_ref, b1_ref, w2_ref, b2_ref, o_ref):
        h = jnp.maximum(jnp.tanh(x_ref[...]) * w1_ref[...] + b1_ref[...], 0.0)
        o_ref[...] = _bdot(h, w2_ref[...]) + b2_ref[...]

    return pl.pallas_call(
        body,
        grid=(nblk,),
        in_specs=[
            pl.BlockSpec((BLK, 1), lambda i: (i, 0)),
            pl.BlockSpec((1, H), lambda i: (0, 0)),
            pl.BlockSpec((1, H), lambda i: (0, 0)),
            pl.BlockSpec((H, H), lambda i: (0, 0)),
            pl.BlockSpec((1, H), lambda i: (0, 0)),
        ],
        out_specs=pl.BlockSpec((BLK, H), lambda i: (i, 0)),
        out_shape=jax.ShapeDtypeStruct((N, H), jnp.float32),
    )(imm, w1, b1.reshape(1, H), w2, b2.reshape(1, H))


def _w_spec(shp):
    return pl.BlockSpec(shp, lambda i: (0, 0))


def _imm_into_table(op_imm, ntab, row0):
    BLK = 2000
    nblk = op_imm.shape[0] // BLK
    blk0 = row0 // BLK

    def body(x_ref, o_ref):
        o_ref[...] = x_ref[...]

    return pl.pallas_call(
        body,
        grid=(nblk,),
        in_specs=[pl.BlockSpec((BLK, H), lambda i: (i, 0))],
        out_specs=pl.BlockSpec((BLK, H), lambda i: (i + blk0, 0)),
        out_shape=jax.ShapeDtypeStruct((ntab, H), jnp.float32),
    )(op_imm)


def _reg_table(table, emb, wr1, br1, wr2, br2):
    BLK = 2000
    nblk = emb.shape[0] // BLK

    def body(tab_ref, emb_ref, wr1_ref, br1_ref, wr2_ref, br2_ref, t_ref):
        h = jnp.maximum(_bdot(emb_ref[...], wr1_ref[...]) + br1_ref[...], 0.0)
        t_ref[...] = _bdot(h, wr2_ref[...]) + br2_ref[...]

    return pl.pallas_call(
        body,
        grid=(nblk,),
        in_specs=[
            pl.BlockSpec(memory_space=pl.ANY),
            pl.BlockSpec((BLK, H), lambda i: (i, 0)),
            _w_spec((H, H)), _w_spec((1, H)), _w_spec((H, H)), _w_spec((1, H)),
        ],
        out_specs=pl.BlockSpec((BLK, H), lambda i: (i, 0)),
        out_shape=jax.ShapeDtypeStruct(table.shape, jnp.float32),
        input_output_aliases={0: 0},
    )(table, emb, wr1, br1.reshape(1, H), wr2, br2.reshape(1, H))


def _mem_into_table(table, a, b, c, d, wm1, bm1, wm2, bm2, row0):
    BLK = 2000
    nblk = a.shape[0] // BLK
    blk0 = row0 // BLK

    def body(tab_ref, a_ref, b_ref, c_ref, d_ref,
             wm1_ref, bm1_ref, wm2_ref, bm2_ref, o_ref):
        acc = (_bdot(a_ref[...], wm1_ref[0:H])
               + _bdot(b_ref[...], wm1_ref[H:2 * H])
               + _bdot(c_ref[...], wm1_ref[2 * H:3 * H])
               + _bdot(d_ref[...], wm1_ref[3 * H:4 * H])
               + bm1_ref[...])
        h = jnp.maximum(acc, 0.0)
        o_ref[...] = _bdot(h, wm2_ref[...]) + bm2_ref[...]

    blkspec = lambda: pl.BlockSpec((BLK, H), lambda i: (i, 0))
    return pl.pallas_call(
        body,
        grid=(nblk,),
        in_specs=[
            pl.BlockSpec(memory_space=pl.ANY),
            blkspec(), blkspec(), blkspec(), blkspec(),
            _w_spec((4 * H, H)), _w_spec((1, H)), _w_spec((H, H)), _w_spec((1, H)),
        ],
        out_specs=pl.BlockSpec((BLK, H), lambda i: (i + blk0, 0)),
        out_shape=jax.ShapeDtypeStruct(table.shape, jnp.float32),
        input_output_aliases={0: 0},
    )(table, a, b, c, d,
      wm1, bm1.reshape(1, H), wm2, bm2.reshape(1, H))


def _final_mlp_half(prev_out, mn_g, g_half, w1, b1, w2, b2,
                    bsz, slen, half, nhalves):
    RB = 64
    BLK = RB * slen
    nblk = bsz // nhalves // RB
    mn0 = half * nblk

    def body(tab_ref, mn_ref, g0_ref, g1_ref, g2_ref, g3_ref,
             w1_ref, b1_ref, w2_ref, b2_ref, o_ref):
        acc = (_bdot(mn_ref[...], w1_ref[0:H])
               + _bdot(g0_ref[...], w1_ref[H:2 * H])
               + _bdot(g1_ref[...], w1_ref[2 * H:3 * H])
               + _bdot(g2_ref[...], w1_ref[3 * H:4 * H])
               + _bdot(g3_ref[...], w1_ref[4 * H:5 * H])
               + b1_ref[...])
        h = jnp.maximum(acc, 0.0)
        o_ref[...] = (_bdot(h, w2_ref[...]) + b2_ref[...]).reshape(RB, slen, H)

    slot = lambda k: pl.BlockSpec((BLK, H), lambda i, k=k: (k * nblk + i, 0))
    first = prev_out is None
    args = [mn_g, g_half, g_half, g_half, g_half,
            w1, b1.reshape(1, H), w2, b2.reshape(1, H)]
    in_specs = [
        pl.BlockSpec((BLK, H), lambda i: (mn0 + i, 0)),
        slot(0), slot(1), slot(2), slot(3),
        _w_spec((5 * H, H)), _w_spec((1, H)), _w_spec((H, H)), _w_spec((1, H)),
    ]
    if first:
        def body1(*refs):
            body(None, *refs)
        kernel_body = body1
        aliases = {}
    else:
        args = [prev_out] + args
        in_specs = [pl.BlockSpec(memory_space=pl.ANY)] + in_specs
        kernel_body = body
        aliases = {0: 0}
    return pl.pallas_call(
        kernel_body,
        grid=(nblk,),
        in_specs=in_specs,
        out_specs=pl.BlockSpec((RB, slen, H), lambda i: (mn0 + i, 0, 0)),
        out_shape=jax.ShapeDtypeStruct((bsz, slen, H), jnp.float32),
        input_output_aliases=aliases,
    )(*args)


def kernel(imm, regs, mem_reg0, mem_reg1, mem_imm0, mem_imm1, mnemic,
           mnemic_idx, operand_idx, emb, W_imm1, b_imm1, W_imm2, b_imm2,
           W_reg1, b_reg1, W_reg2, b_reg2, W_mem1, b_mem1, W_mem2, b_mem2,
           W_ins1, b_ins1, W_ins2, b_ins2):
    nreg = regs.shape[0]
    nimm = imm.shape[0]
    nmem = mem_reg0.shape[0]
    ntab = nreg + nimm + nmem
    regs = regs.astype(jnp.int32)
    bsz, slen = mnemic_idx.shape

    regs_ext = jnp.concatenate(
        [regs, jnp.arange(nreg, ntab, dtype=jnp.int32)])
    nh = bsz // 2 * slen
    oi = operand_idx.reshape(-1, 4)
    opcats = [oi[h * nh:(h + 1) * nh].T.reshape(-1).astype(jnp.int32)
              for h in (0, 1)]

    op_imm = _imm_mlp(imm, W_imm1, b_imm1, W_imm2, b_imm2)
    mn_g = _gather_mn(emb, mnemic.astype(jnp.int32),
                      mnemic_idx.reshape(-1).astype(jnp.int32))
    a, b, c, d = _gather_mem(emb, op_imm, regs,
                             mem_reg0.astype(jnp.int32),
                             mem_reg1.astype(jnp.int32),
                             mem_imm0.astype(jnp.int32),
                             mem_imm1.astype(jnp.int32))
    table1 = _imm_into_table(op_imm, ntab, nreg)
    table2 = _reg_table(table1, emb, W_reg1, b_reg1, W_reg2, b_reg2)
    table3 = _mem_into_table(table2, a, b, c, d,
                             W_mem1, b_mem1, W_mem2, b_mem2, nreg + nimm)
    out = None
    gs = []
    for half in (0, 1):
        gs.append(_gather_final(table3, regs_ext, opcats[half]))
    for half in (0, 1):
        out = _final_mlp_half(out, mn_g, gs[half], W_ins1, b_ins1,
                              W_ins2, b_ins2, bsz, slen, half, 2)
    return out


# speedup vs baseline: 1.0600x; 1.0187x over previous
import functools

import jax
import jax.numpy as jnp
from jax import lax
from jax.experimental import pallas as pl
from jax.experimental.pallas import tpu as pltpu
from jax.experimental.pallas import tpu_sc as plsc

H = 128
CH = 128
NC, NS = 2, 16
NW = NC * NS

_MESH = dict(core_axis_name="c", subcore_axis_name="s")


def _wid():
    return lax.axis_index("s") * NC + lax.axis_index("c")


def _bdot(x, w):
    return jnp.dot(x.astype(jnp.bfloat16), w.astype(jnp.bfloat16),
                   preferred_element_type=jnp.float32)


def _chunk_off(g, nrows):
    return jnp.minimum(g * CH, nrows - CH)


def _egather_batched(tab_hbm, iv, rv, sem, nidx, clamp=None):
    nstream = nidx // CH
    FB = 8
    for t0 in range(0, nstream, FB):
        hs = []
        for b in range(min(FB, nstream - t0)):
            off = (t0 + b) * CH
            src_idx = iv.at[pl.ds(off, CH)]
            hs.append(pltpu.async_copy(tab_hbm.at[src_idx],
                                       rv.at[pl.ds(off, CH)], sem))
        for h in hs:
            h.wait()


def _gather_mn(emb, mnemic, mnidx):
    B = mnidx.shape[0]
    RPW = B // NW
    NB = RPW // 256
    mesh = plsc.VectorSubcoreMesh(**_MESH)

    @functools.partial(
        pl.kernel, mesh=mesh,
        out_type=jax.ShapeDtypeStruct((B, H), jnp.float32),
        scratch_types=[
            pltpu.VMEM((RPW,), jnp.int32),
            pltpu.VMEM((RPW,), jnp.int32),
            pltpu.VMEM((256, H), jnp.float32),
            pltpu.VMEM((256, H), jnp.float32),
            pltpu.SemaphoreType.DMA,
            pltpu.SemaphoreType.DMA,
        ],
    )
    def k(emb_hbm, mn_hbm, idx_hbm, out_hbm, iv, mv, rows0, rows1, s1, s2):
        w = _wid()
        base = w * RPW
        pltpu.sync_copy(idx_hbm.at[pl.ds(base, RPW)], iv)
        _egather_batched(mn_hbm, iv, mv, s1, RPW)

        def body(t, carry):
            hs = []
            for half, rows in ((0, rows0), (1, rows1)):
                o = (2 * t + half) * 256
                hs.append(pltpu.async_copy(
                    emb_hbm.at[mv.at[pl.ds(o, CH)]],
                    rows.at[pl.ds(0, CH)], s2))
                hs.append(pltpu.async_copy(
                    emb_hbm.at[mv.at[pl.ds(o + CH, CH)]],
                    rows.at[pl.ds(CH, CH)], s2))
            for half, rows in ((0, rows0), (1, rows1)):
                o = (2 * t + half) * 256
                hs[2 * half].wait()
                hs[2 * half + 1].wait()
                pltpu.sync_copy(rows, out_hbm.at[pl.ds(base + o, 256)])
            return carry
        lax.fori_loop(0, NB // 2, body, 0)

    return k(emb, mnemic, mnidx)


def _gather_mem(emb, op_imm, regs, mr0, mr1, mi0, mi1):
    B = mr0.shape[0]
    SPAN = ((-(-B // NW)) + 7) // 8 * 8
    NST = (SPAN + CH - 1) // CH
    mesh = plsc.VectorSubcoreMesh(**_MESH)
    sds = jax.ShapeDtypeStruct((B, H), jnp.float32)

    @functools.partial(
        pl.kernel, mesh=mesh,
        out_type=(sds, sds, sds, sds),
        scratch_types=[
            pltpu.VMEM((SPAN,), jnp.int32),
            pltpu.VMEM((SPAN,), jnp.int32),
            pltpu.VMEM((SPAN,), jnp.int32),
            pltpu.VMEM((SPAN,), jnp.int32),
            pltpu.VMEM((SPAN,), jnp.int32),
            pltpu.VMEM((SPAN,), jnp.int32),
            pltpu.VMEM((CH, H), jnp.float32),
            pltpu.VMEM((CH, H), jnp.float32),
            pltpu.VMEM((CH, H), jnp.float32),
            pltpu.VMEM((CH, H), jnp.float32),
            pltpu.SemaphoreType.DMA,
            pltpu.SemaphoreType.DMA,
            pltpu.SemaphoreType.DMA,
        ],
    )
    def k(emb_hbm, imm_hbm, regs_hbm, mr0_hbm, mr1_hbm, mi0_hbm, mi1_hbm,
          a_hbm, b_hbm, c_hbm, d_hbm,
          iv0, iv1, iv2, iv3, rv0, rv1, ra, rb, rc, rd, s1, s2, s3):
        w = _wid()
        base = jnp.minimum(w * SPAN, B - SPAN)
        hs = [pltpu.async_copy(src.at[pl.ds(base, SPAN)], dst, s1)
              for src, dst in ((mr0_hbm, iv0), (mr1_hbm, iv1),
                               (mi0_hbm, iv2), (mi1_hbm, iv3))]
        for h in hs:
            h.wait()
        pairs = [(iv, rv, c) for c in range(NST) for iv, rv in
                 ((iv0, rv0), (iv1, rv1))]
        for t0 in range(0, len(pairs), 8):
            hs = []
            for iv, rv, c in pairs[t0:t0 + 8]:
                o = min(c * CH, SPAN - CH)
                hs.append(pltpu.async_copy(
                    regs_hbm.at[iv.at[pl.ds(o, CH)]],
                    rv.at[pl.ds(o, CH)], s1))
            for h in hs:
                h.wait()

        def body(c, carry):
            o = jnp.minimum(c * CH, SPAN - CH)
            hs = [pltpu.async_copy(tab.at[idx.at[pl.ds(o, CH)]], rows, s2)
                  for tab, idx, rows in ((emb_hbm, rv0, ra), (emb_hbm, rv1, rb),
                                         (imm_hbm, iv2, rc), (imm_hbm, iv3, rd))]
            for h in hs:
                h.wait()
            ws = [pltpu.async_copy(rows, dst.at[pl.ds(base + o, CH)], s3)
                  for rows, dst in ((ra, a_hbm), (rb, b_hbm),
                                    (rc, c_hbm), (rd, d_hbm))]
            for h in ws:
                h.wait()
            return carry
        lax.fori_loop(0, NST, body, 0)

    return k(emb, op_imm, regs, mr0, mr1, mi0, mi1)


def _gather_final(table, regs_ext, opidx):
    B = opidx.shape[0]
    mesh = plsc.VectorSubcoreMesh(**_MESH)
    RPW = B // NW
    NB = RPW // 256

    @functools.partial(
        pl.kernel, mesh=mesh,
        out_type=jax.ShapeDtypeStruct((B, H), jnp.float32),
        scratch_types=[
            pltpu.VMEM((RPW,), jnp.int32),
            pltpu.VMEM((RPW,), jnp.int32),
            pltpu.VMEM((256, H), jnp.float32),
            pltpu.VMEM((256, H), jnp.float32),
            pltpu.SemaphoreType.DMA,
            pltpu.SemaphoreType.DMA,
        ],
    )
    def k(tab_hbm, rext_hbm, idx_hbm, out_hbm, iv, cv, rows0, rows1, s1, s2):
        w = _wid()
        base = w * RPW
        pltpu.sync_copy(idx_hbm.at[pl.ds(base, RPW)], iv)
        _egather_batched(rext_hbm, iv, cv, s1, RPW)

        def body(t, carry):
            hs = []
            for half, rows in ((0, rows0), (1, rows1)):
                o = (2 * t + half) * 256
                hs.append(pltpu.async_copy(
                    tab_hbm.at[cv.at[pl.ds(o, CH)]],
                    rows.at[pl.ds(0, CH)], s2))
                hs.append(pltpu.async_copy(
                    tab_hbm.at[cv.at[pl.ds(o + CH, CH)]],
                    rows.at[pl.ds(CH, CH)], s2))
            for half, rows in ((0, rows0), (1, rows1)):
                o = (2 * t + half) * 256
                hs[2 * half].wait()
                hs[2 * half + 1].wait()
                pltpu.sync_copy(rows, out_hbm.at[pl.ds(base + o, 256)])
            return carry
        lax.fori_loop(0, NB // 2, body, 0)

    return k(table, regs_ext, opidx)


def _imm_mlp(imm, w1, b1, w2, b2):
    N = imm.shape[0]
    BLK = 5000
    nblk = N // BLK

    def body(x_ref, w1_ref, b1_ref, w2_ref, b2_ref, o_ref):
        h = jnp.maximum(jnp.tanh(x_ref[...]) * w1_ref[...] + b1_ref[...], 0.0)
        o_ref[...] = _bdot(h, w2_ref[...]) + b2_ref[...]

    return pl.pallas_call(
        body,
        grid=(nblk,),
        in_specs=[
            pl.BlockSpec((BLK, 1), lambda i: (i, 0)),
            pl.BlockSpec((1, H), lambda i: (0, 0)),
            pl.BlockSpec((1, H), lambda i: (0, 0)),
            pl.BlockSpec((H, H), lambda i: (0, 0)),
            pl.BlockSpec((1, H), lambda i: (0, 0)),
        ],
        out_specs=pl.BlockSpec((BLK, H), lambda i: (i, 0)),
        out_shape=jax.ShapeDtypeStruct((N, H), jnp.float32),
    )(imm, w1, b1.reshape(1, H), w2, b2.reshape(1, H))


def _w_spec(shp):
    return pl.BlockSpec(shp, lambda i: (0, 0))


def _imm_into_table(op_imm, ntab, row0):
    BLK = 5000
    nblk = op_imm.shape[0] // BLK
    blk0 = row0 // BLK

    def body(x_ref, o_ref):
        o_ref[...] = x_ref[...]

    return pl.pallas_call(
        body,
        grid=(nblk,),
        in_specs=[pl.BlockSpec((BLK, H), lambda i: (i, 0))],
        out_specs=pl.BlockSpec((BLK, H), lambda i: (i + blk0, 0)),
        out_shape=jax.ShapeDtypeStruct((ntab, H), jnp.float32),
    )(op_imm)


def _reg_table(table, emb, wr1, br1, wr2, br2):
    BLK = 4000
    nblk = emb.shape[0] // BLK

    def body(tab_ref, emb_ref, wr1_ref, br1_ref, wr2_ref, br2_ref, t_ref):
        h = jnp.maximum(_bdot(emb_ref[...], wr1_ref[...]) + br1_ref[...], 0.0)
        t_ref[...] = _bdot(h, wr2_ref[...]) + br2_ref[...]

    return pl.pallas_call(
        body,
        grid=(nblk,),
        in_specs=[
            pl.BlockSpec(memory_space=pl.ANY),
            pl.BlockSpec((BLK, H), lambda i: (i, 0)),
            _w_spec((H, H)), _w_spec((1, H)), _w_spec((H, H)), _w_spec((1, H)),
        ],
        out_specs=pl.BlockSpec((BLK, H), lambda i: (i, 0)),
        out_shape=jax.ShapeDtypeStruct(table.shape, jnp.float32),
        input_output_aliases={0: 0},
    )(table, emb, wr1, br1.reshape(1, H), wr2, br2.reshape(1, H))


def _mem_into_table(table, a, b, c, d, wm1, bm1, wm2, bm2, row0):
    BLK = 5000
    nblk = a.shape[0] // BLK
    blk0 = row0 // BLK

    def body(tab_ref, a_ref, b_ref, c_ref, d_ref,
             wm1_ref, bm1_ref, wm2_ref, bm2_ref, o_ref):
        acc = (_bdot(a_ref[...], wm1_ref[0:H])
               + _bdot(b_ref[...], wm1_ref[H:2 * H])
               + _bdot(c_ref[...], wm1_ref[2 * H:3 * H])
               + _bdot(d_ref[...], wm1_ref[3 * H:4 * H])
               + bm1_ref[...])
        h = jnp.maximum(acc, 0.0)
        o_ref[...] = _bdot(h, wm2_ref[...]) + bm2_ref[...]

    blkspec = lambda: pl.BlockSpec((BLK, H), lambda i: (i, 0))
    return pl.pallas_call(
        body,
        grid=(nblk,),
        in_specs=[
            pl.BlockSpec(memory_space=pl.ANY),
            blkspec(), blkspec(), blkspec(), blkspec(),
            _w_spec((4 * H, H)), _w_spec((1, H)), _w_spec((H, H)), _w_spec((1, H)),
        ],
        out_specs=pl.BlockSpec((BLK, H), lambda i: (i + blk0, 0)),
        out_shape=jax.ShapeDtypeStruct(table.shape, jnp.float32),
        input_output_aliases={0: 0},
    )(table, a, b, c, d,
      wm1, bm1.reshape(1, H), wm2, bm2.reshape(1, H))


def _final_mlp_half(prev_out, mn_g, g_half, w1, b1, w2, b2,
                    bsz, slen, half, nhalves):
    RB = 64
    BLK = RB * slen
    nblk = bsz // nhalves // RB
    mn0 = half * nblk

    def body(tab_ref, mn_ref, g0_ref, g1_ref, g2_ref, g3_ref,
             w1_ref, b1_ref, w2_ref, b2_ref, o_ref):
        acc = (_bdot(mn_ref[...], w1_ref[0:H])
               + _bdot(g0_ref[...], w1_ref[H:2 * H])
               + _bdot(g1_ref[...], w1_ref[2 * H:3 * H])
               + _bdot(g2_ref[...], w1_ref[3 * H:4 * H])
               + _bdot(g3_ref[...], w1_ref[4 * H:5 * H])
               + b1_ref[...])
        h = jnp.maximum(acc, 0.0)
        o_ref[...] = (_bdot(h, w2_ref[...]) + b2_ref[...]).reshape(RB, slen, H)

    slot = lambda k: pl.BlockSpec((BLK, H), lambda i, k=k: (k * nblk + i, 0))
    first = prev_out is None
    args = [mn_g, g_half, g_half, g_half, g_half,
            w1, b1.reshape(1, H), w2, b2.reshape(1, H)]
    in_specs = [
        pl.BlockSpec((BLK, H), lambda i: (mn0 + i, 0)),
        slot(0), slot(1), slot(2), slot(3),
        _w_spec((5 * H, H)), _w_spec((1, H)), _w_spec((H, H)), _w_spec((1, H)),
    ]
    if first:
        def body1(*refs):
            body(None, *refs)
        kernel_body = body1
        aliases = {}
    else:
        args = [prev_out] + args
        in_specs = [pl.BlockSpec(memory_space=pl.ANY)] + in_specs
        kernel_body = body
        aliases = {0: 0}
    return pl.pallas_call(
        kernel_body,
        grid=(nblk,),
        in_specs=in_specs,
        out_specs=pl.BlockSpec((RB, slen, H), lambda i: (mn0 + i, 0, 0)),
        out_shape=jax.ShapeDtypeStruct((bsz, slen, H), jnp.float32),
        input_output_aliases=aliases,
    )(*args)


def kernel(imm, regs, mem_reg0, mem_reg1, mem_imm0, mem_imm1, mnemic,
           mnemic_idx, operand_idx, emb, W_imm1, b_imm1, W_imm2, b_imm2,
           W_reg1, b_reg1, W_reg2, b_reg2, W_mem1, b_mem1, W_mem2, b_mem2,
           W_ins1, b_ins1, W_ins2, b_ins2):
    nreg = regs.shape[0]
    nimm = imm.shape[0]
    nmem = mem_reg0.shape[0]
    ntab = nreg + nimm + nmem
    regs = regs.astype(jnp.int32)
    bsz, slen = mnemic_idx.shape

    regs_ext = jnp.concatenate(
        [regs, jnp.arange(nreg, ntab, dtype=jnp.int32)])
    nh = bsz // 2 * slen
    oi = operand_idx.reshape(-1, 4)
    opcats = [oi[h * nh:(h + 1) * nh].T.reshape(-1).astype(jnp.int32)
              for h in (0, 1)]

    op_imm = _imm_mlp(imm, W_imm1, b_imm1, W_imm2, b_imm2)
    mn_g = _gather_mn(emb, mnemic.astype(jnp.int32),
                      mnemic_idx.reshape(-1).astype(jnp.int32))
    a, b, c, d = _gather_mem(emb, op_imm, regs,
                             mem_reg0.astype(jnp.int32),
                             mem_reg1.astype(jnp.int32),
                             mem_imm0.astype(jnp.int32),
                             mem_imm1.astype(jnp.int32))
    table1 = _imm_into_table(op_imm, ntab, nreg)
    table2 = _reg_table(table1, emb, W_reg1, b_reg1, W_reg2, b_reg2)
    table3 = _mem_into_table(table2, a, b, c, d,
                             W_mem1, b_mem1, W_mem2, b_mem2, nreg + nimm)
    out = None
    gs = []
    for half in (0, 1):
        gs.append(_gather_final(table3, regs_ext, opcats[half]))
    for half in (0, 1):
        out = _final_mlp_half(out, mn_g, gs[half], W_ins1, b_ins1,
                              W_ins2, b_ins2, bsz, slen, half, 2)
    return out
